# baseline (speedup 1.0000x reference)
import jax
import jax.numpy as jnp
from jax import lax
from jax.experimental import pallas as pl
from jax.experimental.pallas import tpu as pltpu
from jax.experimental.pallas import tpu_sc as plsc

N_CENTER = 100000
D = 64
B = 16384
ALPHA = 0.9
A1 = ALPHA - 1.0

NC = 2
NS = 16
BT = B // NS
HALF = N_CENTER // NC
NCHUNK = 3
CHUNK = 16672
BLK = 64
NJ = BT // BLK
TROWS = 1048
NDB = 17


def _body(feat_hbm, lab_hbm, ctr_hbm, o16_hbm, out_hbm,
          ones16_v, lab_v, idx_v,
          feat0, feat1, accb0, accb1, cnt0, cnt1,
          sem_a, sem_b, sem_c, sem_d, sem_e, sem_f, sem_oa, sem_ob,
          acc_sh, cnt_sh):
    c = lax.axis_index("c")
    s = lax.axis_index("s")

    pltpu.sync_copy(lab_hbm.at[s], lab_v)
    pltpu.sync_copy(o16_hbm, ones16_v)

    feat = [feat0, feat1]
    ctrb = [feat0, feat1]
    accb = [accb0, accb1]
    cntb = [cnt0, cnt1]
    asem = [sem_a, sem_b]
    bsem = [sem_c, sem_d]
    csem = [sem_e, sem_f]
    osem = [sem_oa, sem_ob]

    def chunk_body(chunk, _):
        base = c * HALF + chunk * CHUNK
        crows = jnp.minimum(CHUNK, HALF - chunk * CHUNK)
        tstart = jnp.minimum(s * TROWS, crows - TROWS)

        def dense_rb(b):
            return tstart + min(b * BLK, TROWS - BLK)

        def fill_zero(i, _):
            feat0[i // 4, pl.ds((i % 4) * 16, 16)] = jnp.zeros(
                (16,), jnp.float32)
            return 0
        lax.fori_loop(0, BLK * 4, fill_zero, 0, unroll=4)

        def fill_zero16(i, _):
            cnt0[i, pl.ds(0, 16)] = jnp.zeros((16,), jnp.float32)
            return 0
        lax.fori_loop(0, BLK, fill_zero16, 0, unroll=4)

        zd = []
        for b in range(NDB):
            rb = dense_rb(b)
            zd.append(pltpu.async_copy(
                feat0, acc_sh.at[pl.ds(rb, BLK)], sem_oa))
            zd.append(pltpu.async_copy(
                cnt0, cnt_sh.at[pl.ds(rb, BLK)], sem_ob))
        for d in zd:
            d.wait()
        plsc.subcore_barrier()

        scat = [None, None]

        def fetch(j):
            sl = j % 2
            return pltpu.async_copy(
                feat_hbm.at[pl.ds(s * BT + j * BLK, BLK)], feat[sl], asem[sl])

        pend = fetch(0)
        for j in range(NJ):
            sl = j % 2
            nxt = None
            if j + 1 < NJ:
                if scat[(j + 1) % 2] is not None:
                    scat[(j + 1) % 2][0].wait()
                    scat[(j + 1) % 2][1].wait()
                    scat[(j + 1) % 2] = None
                nxt = fetch(j + 1)
            pend.wait()

            for k in range(BLK // 16):
                v = lab_v[j, pl.ds(k * 16, 16)]
                rel = v - base
                inb = (rel >= 0) & (rel < CHUNK)
                idx_v[j, pl.ds(k * 16, 16)] = jnp.where(inb, rel, CHUNK)

            scat[sl] = (
                pltpu.async_copy(feat[sl], acc_sh.at[idx_v.at[j]],
                                 osem[sl], add=True),
                pltpu.async_copy(ones16_v, cnt_sh.at[idx_v.at[j]],
                                 csem[sl], add=True),
            )
            if nxt is not None:
                pend = nxt
        for d in scat:
            if d is not None:
                d[0].wait()
                d[1].wait()
        plsc.subcore_barrier()

        owr = [None, None]

        def issue_dense(b):
            sl = b % 2
            rb = dense_rb(b)
            return (
                pltpu.async_copy(ctr_hbm.at[pl.ds(base + rb, BLK)],
                                 ctrb[sl], asem[sl]),
                pltpu.async_copy(acc_sh.at[pl.ds(rb, BLK)], accb[sl],
                                 bsem[sl]),
                pltpu.async_copy(cnt_sh.at[pl.ds(rb, BLK)], cntb[sl],
                                 csem[sl]),
            )

        pend = issue_dense(0)
        for b in range(NDB):
            sl = b % 2
            nxt = None
            if b + 1 < NDB:
                if owr[(b + 1) % 2] is not None:
                    owr[(b + 1) % 2].wait()
                    owr[(b + 1) % 2] = None
                nxt = issue_dense(b + 1)
            pend[0].wait()
            pend[1].wait()
            pend[2].wait()

            def combine(r, _):
                cnt = cntb[sl][r, pl.ds(0, 16)]
                scale = 1.0 + A1 * cnt
                for g in range(D // 16):
                    ctr = ctrb[sl][r, pl.ds(g * 16, 16)]
                    acc = accb[sl][r, pl.ds(g * 16, 16)]
                    ctrb[sl][r, pl.ds(g * 16, 16)] = ctr * scale - A1 * acc
                return 0
            lax.fori_loop(0, BLK, combine, 0, unroll=4)

            owr[sl] = pltpu.async_copy(
                ctrb[sl], out_hbm.at[pl.ds(base + dense_rb(b), BLK)], osem[sl])
            if nxt is not None:
                pend = nxt
        for d in owr:
            if d is not None:
                d.wait()

        plsc.subcore_barrier()
        return 0

    lax.fori_loop(0, NCHUNK, chunk_body, 0)


@jax.jit
def _run(features, labels, centers):
    mesh = plsc.VectorSubcoreMesh(core_axis_name="c", subcore_axis_name="s")
    lab3 = labels.reshape(NS, NJ, BLK)
    o16 = jnp.ones((BLK, 16), jnp.float32)
    return pl.kernel(
        _body,
        out_type=jax.ShapeDtypeStruct((N_CENTER, D), jnp.float32),
        mesh=mesh,
        compiler_params=pltpu.CompilerParams(use_tc_tiling_on_sc=False),
        scratch_types=[
            pltpu.VMEM((BLK, 16), jnp.float32),
            pltpu.VMEM((NJ, BLK), jnp.int32),
            pltpu.VMEM((NJ, BLK), jnp.int32),
            pltpu.VMEM((BLK, D), jnp.float32),
            pltpu.VMEM((BLK, D), jnp.float32),
            pltpu.VMEM((BLK, D), jnp.float32),
            pltpu.VMEM((BLK, D), jnp.float32),
            pltpu.VMEM((BLK, 16), jnp.float32),
            pltpu.VMEM((BLK, 16), jnp.float32),
            pltpu.SemaphoreType.DMA,
            pltpu.SemaphoreType.DMA,
            pltpu.SemaphoreType.DMA,
            pltpu.SemaphoreType.DMA,
            pltpu.SemaphoreType.DMA,
            pltpu.SemaphoreType.DMA,
            pltpu.SemaphoreType.DMA,
            pltpu.SemaphoreType.DMA,
            pltpu.VMEM_SHARED((CHUNK + 1, D), jnp.float32),
            pltpu.VMEM_SHARED((CHUNK + 1, 16), jnp.float32),
        ],
    )(features, lab3, centers, o16)


def kernel(features, labels, centers):
    return _run(features, labels, centers)


# speedup vs baseline: 1.1297x; 1.1297x over previous
import jax
import jax.numpy as jnp
from jax import lax
from jax.experimental import pallas as pl
from jax.experimental.pallas import tpu as pltpu
from jax.experimental.pallas import tpu_sc as plsc

N_CENTER = 100000
D = 64
B = 16384
ALPHA = 0.9
A1 = ALPHA - 1.0

NC = 2
NS = 16
BT = B // NS
HALF = N_CENTER // NC
NCHUNK = 3
CHUNK = 16672
BLK = 64
NJ = BT // BLK
TROWS = 1048
NDB = 17


def _body(feat_hbm, lab_hbm, ctr_hbm, o16_hbm, out_hbm,
          ones16_v, lab_v, idx_v,
          feat0, feat1, accb0, accb1, cnt0, cnt1,
          sem_a, sem_b, sem_c, sem_d, sem_e, sem_f, sem_oa, sem_ob,
          acc_sh, cnt_sh):
    c = lax.axis_index("c")
    s = lax.axis_index("s")

    pltpu.sync_copy(lab_hbm.at[s], lab_v)
    pltpu.sync_copy(o16_hbm, ones16_v)

    feat = [feat0, feat1]
    ctrb = [feat0, feat1]
    accb = [accb0, accb1]
    cntb = [cnt0, cnt1]
    asem = [sem_a, sem_b]
    bsem = [sem_c, sem_d]
    csem = [sem_e, sem_f]
    osem = [sem_oa, sem_ob]

    def chunk_body(chunk, _):
        base = c * HALF + chunk * CHUNK
        crows = jnp.minimum(CHUNK, HALF - chunk * CHUNK)
        tstart = jnp.minimum(s * TROWS, crows - TROWS)

        def dense_rb(b):
            return tstart + min(b * BLK, TROWS - BLK)

        def fill_zero(i, _):
            feat0[i // 4, pl.ds((i % 4) * 16, 16)] = jnp.zeros(
                (16,), jnp.float32)
            return 0
        lax.fori_loop(0, BLK * 4, fill_zero, 0, unroll=4)

        def fill_zero16(i, _):
            cnt0[i, pl.ds(0, 16)] = jnp.zeros((16,), jnp.float32)
            return 0
        lax.fori_loop(0, BLK, fill_zero16, 0, unroll=4)

        zd = []
        for b in range(NDB):
            rb = dense_rb(b)
            zd.append(pltpu.async_copy(
                feat0, acc_sh.at[pl.ds(rb, BLK)], sem_oa))
            zd.append(pltpu.async_copy(
                cnt0, cnt_sh.at[pl.ds(rb, BLK)], sem_ob))
        for d in zd:
            d.wait()
        plsc.subcore_barrier()

        scat = [None, None]

        def fetch(j):
            sl = j % 2
            return pltpu.async_copy(
                feat_hbm.at[pl.ds(s * BT + j * BLK, BLK)], feat[sl], asem[sl])

        pend = fetch(0)
        for j in range(NJ):
            sl = j % 2
            nxt = None
            if j + 1 < NJ:
                if scat[(j + 1) % 2] is not None:
                    scat[(j + 1) % 2][0].wait()
                    scat[(j + 1) % 2][1].wait()
                    scat[(j + 1) % 2] = None
                nxt = fetch(j + 1)
            pend.wait()

            for k in range(BLK // 16):
                v = lab_v[j, pl.ds(k * 16, 16)]
                rel = v - base
                inb = (rel >= 0) & (rel < CHUNK)
                dummy = CHUNK + ((j * 4 + k) % 16) * 16 + lax.iota(
                    jnp.int32, 16)
                idx_v[j, pl.ds(k * 16, 16)] = jnp.where(inb, rel, dummy)

            scat[sl] = (
                pltpu.async_copy(feat[sl], acc_sh.at[idx_v.at[j]],
                                 osem[sl], add=True),
                pltpu.async_copy(ones16_v, cnt_sh.at[idx_v.at[j]],
                                 csem[sl], add=True),
            )
            if nxt is not None:
                pend = nxt
        for d in scat:
            if d is not None:
                d[0].wait()
                d[1].wait()
        plsc.subcore_barrier()

        owr = [None, None]

        def issue_dense(b):
            sl = b % 2
            rb = dense_rb(b)
            return (
                pltpu.async_copy(ctr_hbm.at[pl.ds(base + rb, BLK)],
                                 ctrb[sl], asem[sl]),
                pltpu.async_copy(acc_sh.at[pl.ds(rb, BLK)], accb[sl],
                                 bsem[sl]),
                pltpu.async_copy(cnt_sh.at[pl.ds(rb, BLK)], cntb[sl],
                                 csem[sl]),
            )

        pend = issue_dense(0)
        for b in range(NDB):
            sl = b % 2
            nxt = None
            if b + 1 < NDB:
                if owr[(b + 1) % 2] is not None:
                    owr[(b + 1) % 2].wait()
                    owr[(b + 1) % 2] = None
                nxt = issue_dense(b + 1)
            pend[0].wait()
            pend[1].wait()
            pend[2].wait()

            def combine(r, _):
                cnt = cntb[sl][r, pl.ds(0, 16)]
                scale = 1.0 + A1 * cnt
                for g in range(D // 16):
                    ctr = ctrb[sl][r, pl.ds(g * 16, 16)]
                    acc = accb[sl][r, pl.ds(g * 16, 16)]
                    ctrb[sl][r, pl.ds(g * 16, 16)] = ctr * scale - A1 * acc
                return 0
            lax.fori_loop(0, BLK, combine, 0, unroll=4)

            owr[sl] = pltpu.async_copy(
                ctrb[sl], out_hbm.at[pl.ds(base + dense_rb(b), BLK)], osem[sl])
            if nxt is not None:
                pend = nxt
        for d in owr:
            if d is not None:
                d.wait()

        plsc.subcore_barrier()
        return 0

    lax.fori_loop(0, NCHUNK, chunk_body, 0)


@jax.jit
def _run(features, labels, centers):
    mesh = plsc.VectorSubcoreMesh(core_axis_name="c", subcore_axis_name="s")
    lab3 = labels.reshape(NS, NJ, BLK)
    o16 = jnp.ones((BLK, 16), jnp.float32)
    return pl.kernel(
        _body,
        out_type=jax.ShapeDtypeStruct((N_CENTER, D), jnp.float32),
        mesh=mesh,
        compiler_params=pltpu.CompilerParams(use_tc_tiling_on_sc=False),
        scratch_types=[
            pltpu.VMEM((BLK, 16), jnp.float32),
            pltpu.VMEM((NJ, BLK), jnp.int32),
            pltpu.VMEM((NJ, BLK), jnp.int32),
            pltpu.VMEM((BLK, D), jnp.float32),
            pltpu.VMEM((BLK, D), jnp.float32),
            pltpu.VMEM((BLK, D), jnp.float32),
            pltpu.VMEM((BLK, D), jnp.float32),
            pltpu.VMEM((BLK, 16), jnp.float32),
            pltpu.VMEM((BLK, 16), jnp.float32),
            pltpu.SemaphoreType.DMA,
            pltpu.SemaphoreType.DMA,
            pltpu.SemaphoreType.DMA,
            pltpu.SemaphoreType.DMA,
            pltpu.SemaphoreType.DMA,
            pltpu.SemaphoreType.DMA,
            pltpu.SemaphoreType.DMA,
            pltpu.SemaphoreType.DMA,
            pltpu.VMEM_SHARED((CHUNK + 256, D), jnp.float32),
            pltpu.VMEM_SHARED((CHUNK + 256, 16), jnp.float32),
        ],
    )(features, lab3, centers, o16)


def kernel(features, labels, centers):
    return _run(features, labels, centers)
